# Initial kernel scaffold; baseline (speedup 1.0000x reference)
import jax
import jax.numpy as jnp
from jax.experimental import pallas as pl


def kernel(positions, table):
    raise NotImplementedError("write your pallas kernel here")


# speedup vs baseline: 2.3832x; 2.3832x over previous
import functools

import jax
import jax.numpy as jnp
from jax import lax
from jax.experimental import pallas as pl
from jax.experimental.pallas import tpu as pltpu
from jax.experimental.pallas import tpu_sc as plsc

_NC = 2
_NS = 16
_NW = _NC * _NS
_C = 32


@functools.partial(jax.jit, static_argnames=("b_per_w", "d"))
def _sc_gather(pos_flat, table, *, b_per_w, d):
    nchunks = b_per_w // _C
    mesh = plsc.VectorSubcoreMesh(core_axis_name="c", subcore_axis_name="s")

    @functools.partial(
        pl.kernel,
        mesh=mesh,
        out_type=jax.ShapeDtypeStruct((b_per_w * _NW, d), jnp.float32),
        scratch_types=[
            pltpu.VMEM((b_per_w,), jnp.int32),
            pltpu.VMEM((_C, d), jnp.float32),
            pltpu.VMEM((_C, d), jnp.float32),
            pltpu.SemaphoreType.DMA,
            pltpu.SemaphoreType.DMA,
        ],
    )
    def k(pos_hbm, table_hbm, out_hbm, idx_v, buf0, buf1, sem0, sem1):
        wid = lax.axis_index("s") * _NC + lax.axis_index("c")
        base = wid * b_per_w
        pltpu.sync_copy(pos_hbm.at[pl.ds(base, b_per_w)], idx_v)

        bufs = (buf0, buf1)
        sems = (sem0, sem1)

        def start(chunk, b):
            pltpu.async_copy(
                table_hbm.at[idx_v.at[pl.ds(chunk * _C, _C)]], bufs[b], sems[b]
            )

        def wait(b):
            pltpu.make_async_copy(
                table_hbm.at[pl.ds(0, _C)], bufs[b], sems[b]
            ).wait()

        def write(chunk, b):
            pltpu.sync_copy(bufs[b], out_hbm.at[pl.ds(base + chunk * _C, _C)])

        start(0, 0)
        start(1, 1)

        def body(i, carry):
            g = i * 2
            for b in range(2):
                chunk = g + b
                wait(b)
                write(chunk, b)
                start(chunk + 2, b)
            return carry

        lax.fori_loop(0, nchunks // 2 - 1, body, 0)

        for b in range(2):
            chunk = nchunks - 2 + b
            wait(b)
            write(chunk, b)

    return k(pos_flat, table)


def kernel(positions, table):
    b = positions.size
    d = table.shape[1]
    pos_flat = positions.reshape(-1).astype(jnp.int32)
    out = _sc_gather(pos_flat, table, b_per_w=b // _NW, d=d)
    return out.reshape(positions.shape + (d,))
